# baseline (speedup 1.0000x reference)
import functools
import math

import jax
import jax.numpy as jnp
from jax.experimental import pallas as pl
from jax.experimental.pallas import tpu as pltpu


def _layernorm(x, gamma, beta, eps=1e-5):
    mu = jnp.mean(x, axis=-1, keepdims=True)
    var = jnp.mean((x - mu) ** 2, axis=-1, keepdims=True)
    return (x - mu) * jax.lax.rsqrt(var + eps) * gamma + beta


def _encoder_kernel(tok_ref, emb_ref, pe_ref, mask_ref,
                    wqkv_ref, bqkv_ref, wo_ref, bo_ref,
                    w1_ref, b1_ref, w2_ref, b2_ref,
                    g1_ref, be1_ref, g2_ref, be2_ref,
                    gf_ref, bf_ref, o_ref, xbuf, sems,
                    *, B, S, nhead, nlayers):
    D = pe_ref.shape[-1]
    hd = D // nhead
    scale = 1.0 / math.sqrt(hd)
    R = nhead * S

    def issue_all():
        def body(r, carry):
            pltpu.make_async_copy(emb_ref.at[pl.ds(tok_ref[r], 1)],
                                  xbuf.at[pl.ds(r, 1)],
                                  sems.at[r // S]).start()
            return carry
        jax.lax.fori_loop(0, B * S, body, 0)

    def wait(b):
        pltpu.make_async_copy(emb_ref.at[pl.ds(0, S)],
                              xbuf.at[pl.ds(b * S, S)],
                              sems.at[b]).wait()

    row_head = jax.lax.broadcasted_iota(jnp.int32, (R, D), 0) // S
    col_head = jax.lax.broadcasted_iota(jnp.int32, (R, D), 1) // hd
    headmask_bf = (row_head == col_head).astype(jnp.bfloat16)

    mask = mask_ref[...]
    mask_stack = jnp.concatenate([mask] * nhead, axis=0)
    pe = pe_ref[...]

    issue_all()
    SP = 8
    for g in range(B // SP):
        for b in range(SP):
            wait(g * SP + b)
        x = xbuf[g * SP * S:(g + 1) * SP * S] + jnp.concatenate(
            [pe] * SP, axis=0)

        for l in range(nlayers):
            qkv = jnp.dot(x.astype(jnp.bfloat16), wqkv_ref[l],
                          preferred_element_type=jnp.float32) + bqkv_ref[l]

            ctx_parts = []
            for b in range(SP):
                r0 = b * S
                q = (qkv[r0:r0 + S, :D] * scale).astype(jnp.bfloat16)
                k = qkv[r0:r0 + S, D:2 * D].astype(jnp.bfloat16)
                v = qkv[r0:r0 + S, 2 * D:].astype(jnp.bfloat16)

                q8 = jnp.concatenate([q] * nhead, axis=0) * headmask_bf
                s = jax.lax.dot_general(q8, k, (((1,), (1,)), ((), ())),
                                        preferred_element_type=jnp.float32)
                s = s + mask_stack
                s = s - jnp.max(s, axis=-1, keepdims=True)
                e = jnp.exp(s)
                rcp = 1.0 / jnp.sum(e, axis=-1, keepdims=True)

                c = jnp.dot(e.astype(jnp.bfloat16), v,
                            preferred_element_type=jnp.float32)
                ctx = jnp.concatenate(
                    [c[h * S:(h + 1) * S, h * hd:(h + 1) * hd]
                     * rcp[h * S:(h + 1) * S] for h in range(nhead)], axis=1)
                ctx_parts.append(ctx)
            ctx_all = jnp.concatenate(ctx_parts, axis=0)

            attn = jnp.dot(ctx_all.astype(jnp.bfloat16), wo_ref[l],
                           preferred_element_type=jnp.float32) + bo_ref[l]
            x = _layernorm(x + attn, g1_ref[l], be1_ref[l])

            h1 = jnp.maximum(
                jnp.dot(x.astype(jnp.bfloat16), w1_ref[l],
                        preferred_element_type=jnp.float32) + b1_ref[l], 0.0)
            ff = jnp.dot(h1.astype(jnp.bfloat16), w2_ref[l],
                         preferred_element_type=jnp.float32) + b2_ref[l]
            x = _layernorm(x + ff, g2_ref[l], be2_ref[l])

        y = _layernorm(x, gf_ref[...], bf_ref[...])
        for b in range(SP):
            o_ref[g * SP + b] = jnp.mean(y[b * S:(b + 1) * S], axis=0,
                                         keepdims=True)


def kernel(src_tokens, src_mask, embedding, pe, wqkv_t, bqkv, wo_h, bo,
           w1_t, b1, w2_t, b2, g1, be1, g2, be2, norm_g, norm_b):
    B, S = src_tokens.shape
    nlayers, D, _ = wqkv_t.shape
    nhead = wo_h.shape[1]

    pe_s = pe[:S]
    wo_full = wo_h.reshape(nlayers, D, D)
    tokens = src_tokens.reshape(B * S)

    weights = [wqkv_t, bqkv, wo_full, bo, w1_t, b1, w2_t, b2,
               g1, be1, g2, be2, norm_g, norm_b]

    def const_spec(a):
        nd = a.ndim
        return pl.BlockSpec(a.shape, lambda i, *_, nd=nd: (0,) * nd)

    in_specs = [pl.BlockSpec(memory_space=pl.ANY),
                const_spec(pe_s), const_spec(src_mask)]
    in_specs += [const_spec(w) for w in weights]

    out = pl.pallas_call(
        functools.partial(_encoder_kernel, B=B, S=S,
                          nhead=nhead, nlayers=nlayers),
        out_shape=jax.ShapeDtypeStruct((B, 1, D), jnp.float32),
        grid_spec=pltpu.PrefetchScalarGridSpec(
            num_scalar_prefetch=1,
            grid=(1,),
            in_specs=in_specs,
            out_specs=pl.BlockSpec((B, 1, D), lambda i, *_: (0, 0, 0)),
            scratch_shapes=[pltpu.VMEM((B * S, D), jnp.float32),
                            pltpu.SemaphoreType.DMA((B,))],
        ),
        compiler_params=pltpu.CompilerParams(
            dimension_semantics=("arbitrary",),
            disable_bounds_checks=True,
            vmem_limit_bytes=64 * 1024 * 1024),
    )(tokens, embedding, pe_s, src_mask, *weights)
    return out.reshape(B, D)


# speedup vs baseline: 1.0590x; 1.0590x over previous
import functools
import math

import jax
import jax.numpy as jnp
from jax.experimental import pallas as pl
from jax.experimental.pallas import tpu as pltpu


def _layernorm(x, gamma, beta, eps=1e-5):
    mu = jnp.mean(x, axis=-1, keepdims=True)
    var = jnp.mean((x - mu) ** 2, axis=-1, keepdims=True)
    return (x - mu) * jax.lax.rsqrt(var + eps) * gamma + beta


def _encoder_kernel(tok_ref, emb_ref, pe_ref, mask_ref,
                    wqkv_ref, bqkv_ref, wo_ref, bo_ref,
                    w1_ref, b1_ref, w2_ref, b2_ref,
                    g1_ref, be1_ref, g2_ref, be2_ref,
                    gf_ref, bf_ref, o_ref, xbuf, sems,
                    *, B, S, nhead, nlayers):
    D = pe_ref.shape[-1]
    hd = D // nhead
    scale = 1.0 / math.sqrt(hd)
    R = nhead * S

    def issue_all():
        for b in range(B):
            def body(i, carry, b=b):
                r = b * S + i
                pltpu.make_async_copy(emb_ref.at[pl.ds(tok_ref[r], 1)],
                                      xbuf.at[pl.ds(r, 1)],
                                      sems.at[b]).start()
                return carry
            jax.lax.fori_loop(0, S, body, 0, unroll=4)

    def wait(b):
        pltpu.make_async_copy(emb_ref.at[pl.ds(0, S)],
                              xbuf.at[pl.ds(b * S, S)],
                              sems.at[b]).wait()

    row_head = jax.lax.broadcasted_iota(jnp.int32, (R, D), 0) // S
    col_head = jax.lax.broadcasted_iota(jnp.int32, (R, D), 1) // hd
    headmask_bf = (row_head == col_head).astype(jnp.bfloat16)

    mask_bool = mask_ref[...] == 0.0
    mask_stack = jnp.concatenate([mask_bool] * nhead, axis=0)
    pe = pe_ref[...]

    issue_all()
    SP = 2
    for g in range(B // SP):
        for b in range(SP):
            wait(g * SP + b)
        x = xbuf[g * SP * S:(g + 1) * SP * S] + jnp.concatenate(
            [pe] * SP, axis=0)

        for l in range(nlayers):
            qkv = jnp.dot(x.astype(jnp.bfloat16), wqkv_ref[l],
                          preferred_element_type=jnp.float32) + bqkv_ref[l]

            ctx_parts = []
            for b in range(SP):
                r0 = b * S
                q = (qkv[r0:r0 + S, :D] * scale).astype(jnp.bfloat16)
                k = qkv[r0:r0 + S, D:2 * D].astype(jnp.bfloat16)
                v = qkv[r0:r0 + S, 2 * D:].astype(jnp.bfloat16)

                q8 = jnp.concatenate([q] * nhead, axis=0) * headmask_bf
                s = jax.lax.dot_general(q8, k, (((1,), (1,)), ((), ())),
                                        preferred_element_type=jnp.float32)
                s = s - jnp.max(s, axis=-1, keepdims=True)
                e = jnp.where(mask_stack, jnp.exp(s), 0.0)
                rcp = 1.0 / jnp.sum(e, axis=-1, keepdims=True)

                c = jnp.dot(e.astype(jnp.bfloat16), v,
                            preferred_element_type=jnp.float32)
                ctx = jnp.concatenate(
                    [c[h * S:(h + 1) * S, h * hd:(h + 1) * hd]
                     * rcp[h * S:(h + 1) * S] for h in range(nhead)], axis=1)
                ctx_parts.append(ctx)
            ctx_all = jnp.concatenate(ctx_parts, axis=0)

            attn = jnp.dot(ctx_all.astype(jnp.bfloat16), wo_ref[l],
                           preferred_element_type=jnp.float32) + bo_ref[l]
            x = _layernorm(x + attn, g1_ref[l], be1_ref[l])

            h1 = jnp.maximum(
                jnp.dot(x.astype(jnp.bfloat16), w1_ref[l],
                        preferred_element_type=jnp.float32) + b1_ref[l], 0.0)
            ff = jnp.dot(h1.astype(jnp.bfloat16), w2_ref[l],
                         preferred_element_type=jnp.float32) + b2_ref[l]
            x = _layernorm(x + ff, g2_ref[l], be2_ref[l])

        y = _layernorm(x, gf_ref[...], bf_ref[...])
        for b in range(SP):
            o_ref[g * SP + b] = jnp.mean(y[b * S:(b + 1) * S], axis=0,
                                         keepdims=True)


def kernel(src_tokens, src_mask, embedding, pe, wqkv_t, bqkv, wo_h, bo,
           w1_t, b1, w2_t, b2, g1, be1, g2, be2, norm_g, norm_b):
    B, S = src_tokens.shape
    nlayers, D, _ = wqkv_t.shape
    nhead = wo_h.shape[1]

    pe_s = pe[:S]
    wo_full = wo_h.reshape(nlayers, D, D)
    tokens = src_tokens.reshape(B * S)

    weights = [wqkv_t, bqkv, wo_full, bo, w1_t, b1, w2_t, b2,
               g1, be1, g2, be2, norm_g, norm_b]

    def const_spec(a):
        nd = a.ndim
        return pl.BlockSpec(a.shape, lambda i, *_, nd=nd: (0,) * nd)

    in_specs = [pl.BlockSpec(memory_space=pl.ANY),
                const_spec(pe_s), const_spec(src_mask)]
    in_specs += [const_spec(w) for w in weights]

    out = pl.pallas_call(
        functools.partial(_encoder_kernel, B=B, S=S,
                          nhead=nhead, nlayers=nlayers),
        out_shape=jax.ShapeDtypeStruct((B, 1, D), jnp.float32),
        grid_spec=pltpu.PrefetchScalarGridSpec(
            num_scalar_prefetch=1,
            grid=(1,),
            in_specs=in_specs,
            out_specs=pl.BlockSpec((B, 1, D), lambda i, *_: (0, 0, 0)),
            scratch_shapes=[pltpu.VMEM((B * S, D), jnp.float32),
                            pltpu.SemaphoreType.DMA((B,))],
        ),
        compiler_params=pltpu.CompilerParams(
            dimension_semantics=("arbitrary",),
            disable_bounds_checks=True,
            vmem_limit_bytes=64 * 1024 * 1024),
    )(tokens, embedding, pe_s, src_mask, *weights)
    return out.reshape(B, D)


# speedup vs baseline: 1.2843x; 1.2127x over previous
import functools
import math

import jax
import jax.numpy as jnp
from jax.experimental import pallas as pl
from jax.experimental.pallas import tpu as pltpu


def _layernorm(x, gamma, beta, eps=1e-5):
    mu = jnp.mean(x, axis=-1, keepdims=True)
    var = jnp.mean((x - mu) ** 2, axis=-1, keepdims=True)
    return (x - mu) * jax.lax.rsqrt(var + eps) * gamma + beta


def _encoder_kernel(tok_ref, emb_ref, pe_ref, mask_ref,
                    wqkv_ref, bqkv_ref, wo_ref, bo_ref,
                    w1_ref, b1_ref, w2_ref, b2_ref,
                    g1_ref, be1_ref, g2_ref, be2_ref,
                    gf_ref, bf_ref, o_ref, xbuf, sems,
                    *, B, S, nhead, nlayers):
    D = pe_ref.shape[-1]
    hd = D // nhead
    scale = 1.0 / math.sqrt(hd)
    R = nhead * S

    def issue_all():
        for b in range(B):
            def body(i, carry, b=b):
                r = b * S + i
                pltpu.make_async_copy(emb_ref.at[pl.ds(tok_ref[r], 1)],
                                      xbuf.at[pl.ds(r, 1)],
                                      sems.at[b]).start()
                return carry
            jax.lax.fori_loop(0, S, body, 0, unroll=4)

    def wait(b):
        pltpu.make_async_copy(emb_ref.at[pl.ds(0, S)],
                              xbuf.at[pl.ds(b * S, S)],
                              sems.at[b]).wait()

    row_head = jax.lax.broadcasted_iota(jnp.int32, (R, D), 0) // S
    col_head = jax.lax.broadcasted_iota(jnp.int32, (R, D), 1) // hd
    headmask_bf = (row_head == col_head).astype(jnp.bfloat16)

    mask_bool = mask_ref[...] == 0.0
    mask_stack = jnp.concatenate([mask_bool] * nhead, axis=0)
    pe = pe_ref[...]

    issue_all()
    SP = 4
    for g in range(B // SP):
        for b in range(SP):
            wait(g * SP + b)
        x = xbuf[g * SP * S:(g + 1) * SP * S] + jnp.concatenate(
            [pe] * SP, axis=0)

        for l in range(nlayers):
            qkv = jnp.dot(x.astype(jnp.bfloat16), wqkv_ref[l],
                          preferred_element_type=jnp.float32) + bqkv_ref[l]

            ctx_parts = []
            for b in range(SP):
                r0 = b * S
                q = (qkv[r0:r0 + S, :D] * scale).astype(jnp.bfloat16)
                k = qkv[r0:r0 + S, D:2 * D].astype(jnp.bfloat16)
                v = qkv[r0:r0 + S, 2 * D:].astype(jnp.bfloat16)

                q8 = jnp.concatenate([q] * nhead, axis=0) * headmask_bf
                s = jax.lax.dot_general(q8, k, (((1,), (1,)), ((), ())),
                                        preferred_element_type=jnp.float32)
                s = s - jnp.max(s, axis=-1, keepdims=True)
                e = jnp.where(mask_stack, jnp.exp(s), 0.0)
                rcp = 1.0 / jnp.sum(e, axis=-1, keepdims=True)

                c = jnp.dot(e.astype(jnp.bfloat16), v,
                            preferred_element_type=jnp.float32)
                ctx = jnp.concatenate(
                    [c[h * S:(h + 1) * S, h * hd:(h + 1) * hd]
                     * rcp[h * S:(h + 1) * S] for h in range(nhead)], axis=1)
                ctx_parts.append(ctx)
            ctx_all = jnp.concatenate(ctx_parts, axis=0)

            attn = jnp.dot(ctx_all.astype(jnp.bfloat16), wo_ref[l],
                           preferred_element_type=jnp.float32) + bo_ref[l]
            x = _layernorm(x + attn, g1_ref[l], be1_ref[l])

            h1 = jnp.maximum(
                jnp.dot(x.astype(jnp.bfloat16), w1_ref[l],
                        preferred_element_type=jnp.float32) + b1_ref[l], 0.0)
            ff = jnp.dot(h1.astype(jnp.bfloat16), w2_ref[l],
                         preferred_element_type=jnp.float32) + b2_ref[l]
            x = _layernorm(x + ff, g2_ref[l], be2_ref[l])

        y = _layernorm(x, gf_ref[...], bf_ref[...])
        for b in range(SP):
            o_ref[g * SP + b] = jnp.mean(y[b * S:(b + 1) * S], axis=0,
                                         keepdims=True)


def kernel(src_tokens, src_mask, embedding, pe, wqkv_t, bqkv, wo_h, bo,
           w1_t, b1, w2_t, b2, g1, be1, g2, be2, norm_g, norm_b):
    B, S = src_tokens.shape
    nlayers, D, _ = wqkv_t.shape
    nhead = wo_h.shape[1]

    pe_s = pe[:S]
    wo_full = wo_h.reshape(nlayers, D, D)
    tokens = src_tokens.reshape(B * S)

    weights = [wqkv_t, bqkv, wo_full, bo, w1_t, b1, w2_t, b2,
               g1, be1, g2, be2, norm_g, norm_b]

    def const_spec(a):
        nd = a.ndim
        return pl.BlockSpec(a.shape, lambda i, *_, nd=nd: (0,) * nd)

    in_specs = [pl.BlockSpec(memory_space=pl.ANY),
                const_spec(pe_s), const_spec(src_mask)]
    in_specs += [const_spec(w) for w in weights]

    out = pl.pallas_call(
        functools.partial(_encoder_kernel, B=B, S=S,
                          nhead=nhead, nlayers=nlayers),
        out_shape=jax.ShapeDtypeStruct((B, 1, D), jnp.float32),
        grid_spec=pltpu.PrefetchScalarGridSpec(
            num_scalar_prefetch=1,
            grid=(1,),
            in_specs=in_specs,
            out_specs=pl.BlockSpec((B, 1, D), lambda i, *_: (0, 0, 0)),
            scratch_shapes=[pltpu.VMEM((B * S, D), jnp.float32),
                            pltpu.SemaphoreType.DMA((B,))],
        ),
        compiler_params=pltpu.CompilerParams(
            dimension_semantics=("arbitrary",),
            disable_bounds_checks=True,
            vmem_limit_bytes=64 * 1024 * 1024),
    )(tokens, embedding, pe_s, src_mask, *weights)
    return out.reshape(B, D)
